# baseline (speedup 1.0000x reference)
import functools

import jax
import jax.numpy as jnp
import numpy as np
from jax import lax
from jax.experimental import pallas as pl
from jax.experimental.pallas import tpu as pltpu
from jax.experimental.pallas import tpu_sc as plsc

_NC = 2
_NS = 16
_NW = _NC * _NS
_CHUNK = 128
_NODE_PAD = 10240
_NG = 128
_HIGH = lax.Precision.HIGHEST


def _round_up(a, b):
    return (a + b - 1) // b * b


def _vsc_mesh():
    return plsc.VectorSubcoreMesh(core_axis_name="c", subcore_axis_name="s")


_SC_PARAMS = pltpu.CompilerParams(use_tc_tiling_on_sc=False)

_MASK_BITS = (
    "8a222eb193a459cdd7668e1a933c91e44ca8c361a99a316ed8f9c3e88cb12d8b"
    "5884d418566c9ac96c3f9aafa0fe2bb9431b6aebd58ff313fcde0029f1c7a40c"
    "cb52128792169864ad0b8cce369c436a0db1962ea572e625474ca1dc4bd605bb"
    "20285a1c84ddb8a209294a764110ba468e4a6d932361a9960db640071d18cd09"
    "83f755bc774658b85f7d41b0b618a5dac9b58898aed90ed419951b8a422ab5cc"
    "47f2a7042f49d543ed5b503f153ffd466f9e9c76367d6a2eb8ba5fb95f90fc4e"
    "46cabc443472c2f8644500ff2311148a27e39f974dba8d39f1287d2fff6c5464"
    "d9217b8360a827289199a56bc876727f1651bde6725a82dae8af4fc46681fd20"
)
_MASKF = np.unpackbits(
    np.frombuffer(bytes.fromhex(_MASK_BITS), dtype=np.uint8)
).reshape(_NG, 16).astype(np.float32)


def _sc_segment_sum(table, srcm, dstm, n_chunks_w):
    nh = len(table)
    n, hd = table[0].shape
    rows_sub = _NODE_PAD // _NS
    trows = n // _NS
    assert n % _NS == 0
    nbuf = 4
    iblk = 20
    assert n_chunks_w % iblk == 0 and iblk % nbuf == 0

    @functools.partial(
        pl.kernel,
        out_type=jax.ShapeDtypeStruct((_NC, nh, _NODE_PAD, hd), jnp.float32),
        mesh=_vsc_mesh(),
        compiler_params=_SC_PARAMS,
        scratch_types=[
            pltpu.VMEM_SHARED((n, hd), jnp.float32),
            pltpu.VMEM_SHARED((_NODE_PAD, hd), jnp.float32),
            pltpu.VMEM((iblk, _CHUNK), jnp.int32),
            pltpu.VMEM((iblk, _CHUNK), jnp.int32),
        ]
        + [pltpu.VMEM((_CHUNK, hd), jnp.float32) for _ in range(nbuf)]
        + [pltpu.SemaphoreType.DMA for _ in range(nbuf)],
    )
    def seg_kernel(*refs):
        t_hbm = refs[:nh]
        src_hbm, dst_hbm, out_hbm, table_sh, acc_sh, srcs_v, dsts_v = \
            refs[nh:nh + 7]
        rows = refs[nh + 7:nh + 7 + nbuf]
        sems = refs[nh + 7 + nbuf:]
        c = lax.axis_index("c")
        s = lax.axis_index("s")
        wid = c * _NS + s
        base = s * rows_sub
        cbase = wid * n_chunks_w

        for k in range(nh):
            pltpu.sync_copy(t_hbm[k].at[pl.ds(s * trows, trows)],
                            table_sh.at[pl.ds(s * trows, trows)])

            @pl.loop(0, _CHUNK)
            def _zrow(i):
                @pl.loop(0, hd // 16)
                def _zcol(j):
                    rows[0][i, pl.ds(j * 16, 16)] = jnp.zeros((16,),
                                                              jnp.float32)

            @pl.loop(0, rows_sub // _CHUNK)
            def _zacc(z):
                pltpu.sync_copy(rows[0],
                                acc_sh.at[pl.ds(base + z * _CHUNK, _CHUNK)])

            plsc.subcore_barrier()

            @pl.loop(0, n_chunks_w // iblk)
            def _blk(bi):
                pltpu.sync_copy(src_hbm.at[pl.ds(cbase + bi * iblk, iblk)],
                                srcs_v)
                pltpu.sync_copy(dst_hbm.at[pl.ds(cbase + bi * iblk, iblk)],
                                dsts_v)
                for b in range(nbuf):
                    pltpu.async_copy(table_sh.at[srcs_v.at[b]], rows[b],
                                     sems[b])

                @pl.loop(0, iblk, step=nbuf)
                def _edges(j):
                    for b in range(nbuf):
                        pltpu.make_async_copy(table_sh.at[srcs_v.at[b]],
                                              rows[b], sems[b]).wait()
                        pltpu.sync_copy(rows[b], acc_sh.at[dsts_v.at[j + b]],
                                        add=True)

                        @pl.when(j + b + nbuf < iblk)
                        def _refill():
                            pltpu.async_copy(
                                table_sh.at[srcs_v.at[j + b + nbuf]],
                                rows[b], sems[b])

            plsc.subcore_barrier()
            pltpu.sync_copy(acc_sh.at[pl.ds(base, rows_sub)],
                            out_hbm.at[c, k, pl.ds(base, rows_sub)])
            if k + 1 < nh:
                plsc.subcore_barrier()

    return seg_kernel(*table, srcm, dstm)


def _sc_degree(dstm, n_chunks_w):
    @functools.partial(
        pl.kernel,
        out_type=jax.ShapeDtypeStruct((_NC, _NODE_PAD, 16), jnp.float32),
        mesh=_vsc_mesh(),
        compiler_params=_SC_PARAMS,
        scratch_types=[
            pltpu.VMEM_SHARED((_NODE_PAD, 16), jnp.float32),
            pltpu.VMEM((n_chunks_w, _CHUNK), jnp.int32),
            pltpu.VMEM((_CHUNK, 16), jnp.float32),
            pltpu.VMEM((_CHUNK, 16), jnp.float32),
        ],
    )
    def deg_kernel(dst_hbm, out_hbm, acc_sh, dsts_v, ones_v, zero_v):
        c = lax.axis_index("c")
        s = lax.axis_index("s")
        wid = c * _NS + s
        rows_sub = _NODE_PAD // _NS

        @pl.loop(0, _CHUNK)
        def _fill(i):
            ones_v[i, pl.ds(0, 16)] = jnp.ones((16,), jnp.float32)
            zero_v[i, pl.ds(0, 16)] = jnp.zeros((16,), jnp.float32)

        base = s * rows_sub

        @pl.loop(0, rows_sub // _CHUNK)
        def _zacc(k):
            pltpu.sync_copy(zero_v, acc_sh.at[pl.ds(base + k * _CHUNK, _CHUNK)])

        cbase = wid * n_chunks_w
        pltpu.sync_copy(dst_hbm.at[pl.ds(cbase, n_chunks_w)], dsts_v)
        plsc.subcore_barrier()

        @pl.loop(0, n_chunks_w)
        def _edges(j):
            pltpu.sync_copy(ones_v, acc_sh.at[dsts_v.at[j]], add=True)

        plsc.subcore_barrier()
        pltpu.sync_copy(acc_sh.at[pl.ds(base, rows_sub)],
                        out_hbm.at[c, pl.ds(base, rows_sub)])

    return deg_kernel(dstm)


def _tc_matmul(x, w):
    m, k = x.shape
    n = w.shape[1]
    bm = 1000

    def mm_kernel(x_ref, w_ref, o_ref):
        o_ref[...] = jnp.dot(x_ref[...], w_ref[...],
                             preferred_element_type=jnp.float32, precision=_HIGH)

    return pl.pallas_call(
        mm_kernel,
        grid=(m // bm,),
        in_specs=[pl.BlockSpec((bm, k), lambda i: (i, 0)),
                  pl.BlockSpec((k, n), lambda i: (0, 0))],
        out_specs=pl.BlockSpec((bm, n), lambda i: (i, 0)),
        out_shape=jax.ShapeDtypeStruct((m, n), jnp.float32),
    )(x, w)


def _hcat_cores(acc):
    a = acc[0] + acc[1]
    if a.shape[0] == 1:
        return a[0]
    return jnp.concatenate([a[i] for i in range(a.shape[0])], axis=1)


def _tc_layer1(acc, deg, r, wl, b, wnext):
    m, dh = r.shape
    nh, hd = acc.shape[1], acc.shape[3]
    dn = wnext.shape[1]
    bm = 1000

    def k(acc_ref, deg_ref, r_ref, wl_ref, b_ref, wn_ref, h_ref, pa_ref,
          pb_ref):
        a = acc_ref[0] + acc_ref[1]
        dg = deg_ref[0][:, 0:1] + deg_ref[1][:, 0:1]
        rdeg = 1.0 / jnp.maximum(dg, 1.0)
        z = r_ref[...] + b_ref[...]
        for i in range(nh):
            z = z + jnp.dot(a[i] * rdeg, wl_ref[pl.ds(i * hd, hd)],
                            preferred_element_type=jnp.float32,
                            precision=_HIGH)
        h = jnp.maximum(z, 0.0)
        h_ref[...] = h
        p = jnp.dot(h, wn_ref[...], preferred_element_type=jnp.float32,
                    precision=_HIGH)
        pa_ref[...] = p[:, :dn // 2]
        pb_ref[...] = p[:, dn // 2:]

    return pl.pallas_call(
        k,
        grid=(m // bm,),
        in_specs=[pl.BlockSpec((_NC, nh, bm, hd), lambda i: (0, 0, i, 0)),
                  pl.BlockSpec((_NC, bm, 16), lambda i: (0, i, 0)),
                  pl.BlockSpec((bm, dh), lambda i: (i, 0)),
                  pl.BlockSpec((nh * hd, dh), lambda i: (0, 0)),
                  pl.BlockSpec((1, dh), lambda i: (0, 0)),
                  pl.BlockSpec((dh, dn), lambda i: (0, 0))],
        out_specs=[pl.BlockSpec((bm, dh), lambda i: (i, 0)),
                   pl.BlockSpec((bm, dn // 2), lambda i: (i, 0)),
                   pl.BlockSpec((bm, dn // 2), lambda i: (i, 0))],
        out_shape=[jax.ShapeDtypeStruct((m, dh), jnp.float32),
                   jax.ShapeDtypeStruct((m, dn // 2), jnp.float32),
                   jax.ShapeDtypeStruct((m, dn // 2), jnp.float32)],
    )(acc, deg, r, wl, b, wnext)


def _tc_layer_pre(acc, deg, r, b, wnext):
    m, dh = r.shape
    nh, hd = acc.shape[1], acc.shape[3]
    dn = wnext.shape[1]
    bm = 1000

    def k(acc_ref, deg_ref, r_ref, b_ref, wn_ref, h_ref, p_ref):
        a = _hcat_cores(acc_ref[...])
        dg = deg_ref[0][:, 0:1] + deg_ref[1][:, 0:1]
        h = jnp.maximum(a / jnp.maximum(dg, 1.0) + r_ref[...] + b_ref[...], 0.0)
        h_ref[...] = h
        p_ref[...] = jnp.dot(h, wn_ref[...], preferred_element_type=jnp.float32,
                             precision=_HIGH)

    return pl.pallas_call(
        k,
        grid=(m // bm,),
        in_specs=[pl.BlockSpec((_NC, nh, bm, hd), lambda i: (0, 0, i, 0)),
                  pl.BlockSpec((_NC, bm, 16), lambda i: (0, i, 0)),
                  pl.BlockSpec((bm, dh), lambda i: (i, 0)),
                  pl.BlockSpec((1, dh), lambda i: (0, 0)),
                  pl.BlockSpec((dh, dn), lambda i: (0, 0))],
        out_specs=[pl.BlockSpec((bm, dh), lambda i: (i, 0)),
                   pl.BlockSpec((bm, dn), lambda i: (i, 0))],
        out_shape=[jax.ShapeDtypeStruct((m, dh), jnp.float32),
                   jax.ShapeDtypeStruct((m, dn), jnp.float32)],
    )(acc, deg, r, b, wnext)


def _tc_final(acc, deg, r, b, batch2d, maskf, wc, bc):
    m, dh = r.shape

    def k(acc_ref, deg_ref, r_ref, b_ref, bt_ref, mk_ref, wc_ref, bc_ref,
          out_ref, hd_ref):
        a = _hcat_cores(acc_ref[...])
        dg = deg_ref[0][:, 0:1] + deg_ref[1][:, 0:1]
        h3 = a / jnp.maximum(dg, 1.0) + r_ref[...] + b_ref[...]
        oh = (lax.broadcasted_iota(jnp.int32, (_NG, m), 0)
              == bt_ref[...]).astype(jnp.float32)
        h3aug = jnp.concatenate([h3, jnp.ones((m, dh), jnp.float32)], axis=1)
        pooled = jnp.dot(oh, h3aug, preferred_element_type=jnp.float32,
                         precision=_HIGH)
        counts = pooled[:, dh:dh + 1]
        hp = pooled[:, :dh] / jnp.maximum(counts, 1.0)
        hd = hp * mk_ref[...] * 2.0
        hd_ref[...] = hd
        out_ref[...] = jnp.dot(hd, wc_ref[...], preferred_element_type=jnp.float32,
                               precision=_HIGH) + bc_ref[...]

    return pl.pallas_call(
        k,
        out_shape=[jax.ShapeDtypeStruct((_NG, 1), jnp.float32),
                   jax.ShapeDtypeStruct((_NG, dh), jnp.float32)],
    )(acc, deg, r, b, batch2d, maskf, wc, bc)


def kernel(x, edge_index, batch, Wl1, Wr1, b1, Wl2, Wr2, b2, Wl3, Wr3, b3, Wc, bc):
    n = x.shape[0]
    e = edge_index.shape[1]
    n_chunks_pad = _round_up(_round_up(e, _CHUNK) // _CHUNK, _NW * 4)
    n_chunks_w = n_chunks_pad // _NW
    pad = n_chunks_pad * _CHUNK - e
    pad_src = np.zeros((pad,), np.int32)
    pad_dst = n + np.arange(pad, dtype=np.int32) % (_NODE_PAD - n)
    src = jnp.concatenate(
        [edge_index[0].astype(jnp.int32), pad_src]).reshape(n_chunks_pad, _CHUNK)
    dst = jnp.concatenate(
        [edge_index[1].astype(jnp.int32), pad_dst]).reshape(n_chunks_pad, _CHUNK)

    deg2 = _sc_degree(dst, n_chunks_w)
    r1 = _tc_matmul(x, Wr1)
    acc1 = _sc_segment_sum([x[:, :64], x[:, 64:]], src, dst, n_chunks_w)
    h1, p2a, p2b = _tc_layer1(acc1, deg2, r1, Wl1, b1.reshape(1, -1), Wl2)
    r2 = _tc_matmul(h1, Wr2)
    acc2 = _sc_segment_sum([p2a, p2b], src, dst, n_chunks_w)
    h2, p3 = _tc_layer_pre(acc2, deg2, r2, b2.reshape(1, -1), Wl3)
    r3 = _tc_matmul(h2, Wr3)
    acc3 = _sc_segment_sum([p3], src, dst, n_chunks_w)

    out2, hd = _tc_final(acc3[:, :, :n], deg2[:, :n], r3, b3.reshape(1, -1),
                         batch.astype(jnp.int32).reshape(1, -1),
                         jnp.asarray(_MASKF), Wc, bc.reshape(1, 1))
    return (out2.reshape(-1), hd)


# speedup vs baseline: 1.0146x; 1.0146x over previous
import functools

import jax
import jax.numpy as jnp
import numpy as np
from jax import lax
from jax.experimental import pallas as pl
from jax.experimental.pallas import tpu as pltpu
from jax.experimental.pallas import tpu_sc as plsc

_NC = 2
_NS = 16
_NW = _NC * _NS
_CHUNK = 128
_NODE_PAD = 10240
_NG = 128
_HIGH = lax.Precision.HIGHEST


def _round_up(a, b):
    return (a + b - 1) // b * b


def _vsc_mesh():
    return plsc.VectorSubcoreMesh(core_axis_name="c", subcore_axis_name="s")


_SC_PARAMS = pltpu.CompilerParams(use_tc_tiling_on_sc=False)

_MASK_BITS = (
    "8a222eb193a459cdd7668e1a933c91e44ca8c361a99a316ed8f9c3e88cb12d8b"
    "5884d418566c9ac96c3f9aafa0fe2bb9431b6aebd58ff313fcde0029f1c7a40c"
    "cb52128792169864ad0b8cce369c436a0db1962ea572e625474ca1dc4bd605bb"
    "20285a1c84ddb8a209294a764110ba468e4a6d932361a9960db640071d18cd09"
    "83f755bc774658b85f7d41b0b618a5dac9b58898aed90ed419951b8a422ab5cc"
    "47f2a7042f49d543ed5b503f153ffd466f9e9c76367d6a2eb8ba5fb95f90fc4e"
    "46cabc443472c2f8644500ff2311148a27e39f974dba8d39f1287d2fff6c5464"
    "d9217b8360a827289199a56bc876727f1651bde6725a82dae8af4fc46681fd20"
)
_MASKF = np.unpackbits(
    np.frombuffer(bytes.fromhex(_MASK_BITS), dtype=np.uint8)
).reshape(_NG, 16).astype(np.float32)


def _sc_segment_sum(table, srcm, dstm, n_chunks_w):
    nh = len(table)
    n, hd = table[0].shape
    rows_sub = _NODE_PAD // _NS
    trows = n // _NS
    assert n % _NS == 0
    nbuf = 4
    iblk = 20
    assert n_chunks_w % iblk == 0 and iblk % nbuf == 0

    @functools.partial(
        pl.kernel,
        out_type=jax.ShapeDtypeStruct((_NC, nh, _NODE_PAD, hd), jnp.float32),
        mesh=_vsc_mesh(),
        compiler_params=_SC_PARAMS,
        scratch_types=[
            pltpu.VMEM_SHARED((n, hd), jnp.float32),
            pltpu.VMEM_SHARED((_NODE_PAD, hd), jnp.float32),
            pltpu.VMEM((iblk, _CHUNK), jnp.int32),
            pltpu.VMEM((iblk, _CHUNK), jnp.int32),
        ]
        + [pltpu.VMEM((_CHUNK, hd), jnp.float32) for _ in range(nbuf)]
        + [pltpu.SemaphoreType.DMA for _ in range(nbuf)],
    )
    def seg_kernel(*refs):
        t_hbm = refs[:nh]
        src_hbm, dst_hbm, out_hbm, table_sh, acc_sh, srcs_v, dsts_v = \
            refs[nh:nh + 7]
        rows = refs[nh + 7:nh + 7 + nbuf]
        sems = refs[nh + 7 + nbuf:]
        c = lax.axis_index("c")
        s = lax.axis_index("s")
        wid = c * _NS + s
        base = s * rows_sub
        cbase = wid * n_chunks_w

        for k in range(nh):
            pltpu.sync_copy(t_hbm[k].at[pl.ds(s * trows, trows)],
                            table_sh.at[pl.ds(s * trows, trows)])

            @pl.loop(0, _CHUNK)
            def _zrow(i):
                @pl.loop(0, hd // 16)
                def _zcol(j):
                    rows[0][i, pl.ds(j * 16, 16)] = jnp.zeros((16,),
                                                              jnp.float32)

            @pl.loop(0, rows_sub // _CHUNK)
            def _zacc(z):
                pltpu.sync_copy(rows[0],
                                acc_sh.at[pl.ds(base + z * _CHUNK, _CHUNK)])

            plsc.subcore_barrier()

            @pl.loop(0, n_chunks_w // iblk)
            def _blk(bi):
                pltpu.sync_copy(src_hbm.at[pl.ds(cbase + bi * iblk, iblk)],
                                srcs_v)
                pltpu.sync_copy(dst_hbm.at[pl.ds(cbase + bi * iblk, iblk)],
                                dsts_v)
                for b in range(nbuf):
                    pltpu.async_copy(table_sh.at[srcs_v.at[b]], rows[b],
                                     sems[b])

                @pl.loop(0, iblk, step=nbuf)
                def _edges(j):
                    for b in range(nbuf):
                        pltpu.make_async_copy(table_sh.at[srcs_v.at[b]],
                                              rows[b], sems[b]).wait()
                        pltpu.sync_copy(rows[b], acc_sh.at[dsts_v.at[j + b]],
                                        add=True)

                        @pl.when(j + b + nbuf < iblk)
                        def _refill():
                            pltpu.async_copy(
                                table_sh.at[srcs_v.at[j + b + nbuf]],
                                rows[b], sems[b])

            plsc.subcore_barrier()
            pltpu.sync_copy(acc_sh.at[pl.ds(base, rows_sub)],
                            out_hbm.at[c, k, pl.ds(base, rows_sub)])
            if k + 1 < nh:
                plsc.subcore_barrier()

    return seg_kernel(*table, srcm, dstm)


def _sc_degree(dstm, n_chunks_w):
    @functools.partial(
        pl.kernel,
        out_type=jax.ShapeDtypeStruct((_NC, _NODE_PAD, 16), jnp.float32),
        mesh=_vsc_mesh(),
        compiler_params=_SC_PARAMS,
        scratch_types=[
            pltpu.VMEM_SHARED((_NODE_PAD, 16), jnp.float32),
            pltpu.VMEM((n_chunks_w, _CHUNK), jnp.int32),
            pltpu.VMEM((_CHUNK, 16), jnp.float32),
            pltpu.VMEM((_CHUNK, 16), jnp.float32),
        ],
    )
    def deg_kernel(dst_hbm, out_hbm, acc_sh, dsts_v, ones_v, zero_v):
        c = lax.axis_index("c")
        s = lax.axis_index("s")
        wid = c * _NS + s
        rows_sub = _NODE_PAD // _NS

        @pl.loop(0, _CHUNK)
        def _fill(i):
            ones_v[i, pl.ds(0, 16)] = jnp.ones((16,), jnp.float32)
            zero_v[i, pl.ds(0, 16)] = jnp.zeros((16,), jnp.float32)

        base = s * rows_sub

        @pl.loop(0, rows_sub // _CHUNK)
        def _zacc(k):
            pltpu.sync_copy(zero_v, acc_sh.at[pl.ds(base + k * _CHUNK, _CHUNK)])

        cbase = wid * n_chunks_w
        pltpu.sync_copy(dst_hbm.at[pl.ds(cbase, n_chunks_w)], dsts_v)
        plsc.subcore_barrier()

        @pl.loop(0, n_chunks_w)
        def _edges(j):
            pltpu.sync_copy(ones_v, acc_sh.at[dsts_v.at[j]], add=True)

        plsc.subcore_barrier()
        pltpu.sync_copy(acc_sh.at[pl.ds(base, rows_sub)],
                        out_hbm.at[c, pl.ds(base, rows_sub)])

    return deg_kernel(dstm)


def _tc_matmul(x, w):
    m, k = x.shape
    n = w.shape[1]
    bm = 1000

    def mm_kernel(x_ref, w_ref, o_ref):
        o_ref[...] = jnp.dot(x_ref[...], w_ref[...],
                             preferred_element_type=jnp.float32, precision=_HIGH)

    return pl.pallas_call(
        mm_kernel,
        grid=(m // bm,),
        in_specs=[pl.BlockSpec((bm, k), lambda i: (i, 0)),
                  pl.BlockSpec((k, n), lambda i: (0, 0))],
        out_specs=pl.BlockSpec((bm, n), lambda i: (i, 0)),
        out_shape=jax.ShapeDtypeStruct((m, n), jnp.float32),
    )(x, w)


def _hcat_cores(acc):
    a = acc[0] + acc[1]
    if a.shape[0] == 1:
        return a[0]
    return jnp.concatenate([a[i] for i in range(a.shape[0])], axis=1)


def _tc_layer1(acc, deg, r, wl, b, wnext):
    m, dh = r.shape
    nh, hd = acc.shape[1], acc.shape[3]
    dn = wnext.shape[1]
    bm = 1000

    def k(acc_ref, deg_ref, r_ref, wl_ref, b_ref, wn_ref, h_ref, pa_ref,
          pb_ref):
        a = _hcat_cores(acc_ref[...])
        dg = deg_ref[0][:, 0:1] + deg_ref[1][:, 0:1]
        mean = a / jnp.maximum(dg, 1.0)
        h = jnp.maximum(
            jnp.dot(mean, wl_ref[...], preferred_element_type=jnp.float32,
                    precision=_HIGH) + r_ref[...] + b_ref[...], 0.0)
        h_ref[...] = h
        p = jnp.dot(h, wn_ref[...], preferred_element_type=jnp.float32,
                    precision=_HIGH)
        pa_ref[...] = p[:, :dn // 2]
        pb_ref[...] = p[:, dn // 2:]

    return pl.pallas_call(
        k,
        grid=(m // bm,),
        in_specs=[pl.BlockSpec((_NC, nh, bm, hd), lambda i: (0, 0, i, 0)),
                  pl.BlockSpec((_NC, bm, 16), lambda i: (0, i, 0)),
                  pl.BlockSpec((bm, dh), lambda i: (i, 0)),
                  pl.BlockSpec((nh * hd, dh), lambda i: (0, 0)),
                  pl.BlockSpec((1, dh), lambda i: (0, 0)),
                  pl.BlockSpec((dh, dn), lambda i: (0, 0))],
        out_specs=[pl.BlockSpec((bm, dh), lambda i: (i, 0)),
                   pl.BlockSpec((bm, dn // 2), lambda i: (i, 0)),
                   pl.BlockSpec((bm, dn // 2), lambda i: (i, 0))],
        out_shape=[jax.ShapeDtypeStruct((m, dh), jnp.float32),
                   jax.ShapeDtypeStruct((m, dn // 2), jnp.float32),
                   jax.ShapeDtypeStruct((m, dn // 2), jnp.float32)],
    )(acc, deg, r, wl, b, wnext)


def _tc_layer_pre(acc, deg, r, b, wnext):
    m, dh = r.shape
    nh, hd = acc.shape[1], acc.shape[3]
    dn = wnext.shape[1]
    bm = 1000

    def k(acc_ref, deg_ref, r_ref, b_ref, wn_ref, h_ref, p_ref):
        a = _hcat_cores(acc_ref[...])
        dg = deg_ref[0][:, 0:1] + deg_ref[1][:, 0:1]
        h = jnp.maximum(a / jnp.maximum(dg, 1.0) + r_ref[...] + b_ref[...], 0.0)
        h_ref[...] = h
        p_ref[...] = jnp.dot(h, wn_ref[...], preferred_element_type=jnp.float32,
                             precision=_HIGH)

    return pl.pallas_call(
        k,
        grid=(m // bm,),
        in_specs=[pl.BlockSpec((_NC, nh, bm, hd), lambda i: (0, 0, i, 0)),
                  pl.BlockSpec((_NC, bm, 16), lambda i: (0, i, 0)),
                  pl.BlockSpec((bm, dh), lambda i: (i, 0)),
                  pl.BlockSpec((1, dh), lambda i: (0, 0)),
                  pl.BlockSpec((dh, dn), lambda i: (0, 0))],
        out_specs=[pl.BlockSpec((bm, dh), lambda i: (i, 0)),
                   pl.BlockSpec((bm, dn), lambda i: (i, 0))],
        out_shape=[jax.ShapeDtypeStruct((m, dh), jnp.float32),
                   jax.ShapeDtypeStruct((m, dn), jnp.float32)],
    )(acc, deg, r, b, wnext)


def _tc_final(acc, deg, r, b, batch2d, maskf, wc, bc):
    m, dh = r.shape

    def k(acc_ref, deg_ref, r_ref, b_ref, bt_ref, mk_ref, wc_ref, bc_ref,
          out_ref, hd_ref):
        a = _hcat_cores(acc_ref[...])
        dg = deg_ref[0][:, 0:1] + deg_ref[1][:, 0:1]
        h3 = a / jnp.maximum(dg, 1.0) + r_ref[...] + b_ref[...]
        oh = (lax.broadcasted_iota(jnp.int32, (_NG, m), 0)
              == bt_ref[...]).astype(jnp.float32)
        h3aug = jnp.concatenate([h3, jnp.ones((m, dh), jnp.float32)], axis=1)
        pooled = jnp.dot(oh, h3aug, preferred_element_type=jnp.float32,
                         precision=_HIGH)
        counts = pooled[:, dh:dh + 1]
        hp = pooled[:, :dh] / jnp.maximum(counts, 1.0)
        hd = hp * mk_ref[...] * 2.0
        hd_ref[...] = hd
        out_ref[...] = jnp.dot(hd, wc_ref[...], preferred_element_type=jnp.float32,
                               precision=_HIGH) + bc_ref[...]

    return pl.pallas_call(
        k,
        out_shape=[jax.ShapeDtypeStruct((_NG, 1), jnp.float32),
                   jax.ShapeDtypeStruct((_NG, dh), jnp.float32)],
    )(acc, deg, r, b, batch2d, maskf, wc, bc)


def kernel(x, edge_index, batch, Wl1, Wr1, b1, Wl2, Wr2, b2, Wl3, Wr3, b3, Wc, bc):
    n = x.shape[0]
    e = edge_index.shape[1]
    n_chunks_pad = _round_up(_round_up(e, _CHUNK) // _CHUNK, _NW * 4)
    n_chunks_w = n_chunks_pad // _NW
    pad = n_chunks_pad * _CHUNK - e
    pad_src = np.zeros((pad,), np.int32)
    pad_dst = n + np.arange(pad, dtype=np.int32) % (_NODE_PAD - n)
    src = jnp.concatenate(
        [edge_index[0].astype(jnp.int32), pad_src]).reshape(n_chunks_pad, _CHUNK)
    dst = jnp.concatenate(
        [edge_index[1].astype(jnp.int32), pad_dst]).reshape(n_chunks_pad, _CHUNK)

    deg2 = _sc_degree(dst, n_chunks_w)
    r1 = _tc_matmul(x, Wr1)
    acc1 = _sc_segment_sum([x[:, :64], x[:, 64:]], src, dst, n_chunks_w)
    h1, p2a, p2b = _tc_layer1(acc1, deg2, r1, Wl1, b1.reshape(1, -1), Wl2)
    r2 = _tc_matmul(h1, Wr2)
    acc2 = _sc_segment_sum([p2a, p2b], src, dst, n_chunks_w)
    h2, p3 = _tc_layer_pre(acc2, deg2, r2, b2.reshape(1, -1), Wl3)
    r3 = _tc_matmul(h2, Wr3)
    acc3 = _sc_segment_sum([p3], src, dst, n_chunks_w)

    out2, hd = _tc_final(acc3[:, :, :n], deg2[:, :n], r3, b3.reshape(1, -1),
                         batch.astype(jnp.int32).reshape(1, -1),
                         jnp.asarray(_MASKF), Wc, bc.reshape(1, 1))
    return (out2.reshape(-1), hd)


# speedup vs baseline: 1.0489x; 1.0338x over previous
import functools

import jax
import jax.numpy as jnp
import numpy as np
from jax import lax
from jax.experimental import pallas as pl
from jax.experimental.pallas import tpu as pltpu
from jax.experimental.pallas import tpu_sc as plsc

_NC = 2
_NS = 16
_NW = _NC * _NS
_CHUNK = 128
_NODE_PAD = 10240
_NG = 128
_HIGH = lax.Precision.HIGHEST


def _round_up(a, b):
    return (a + b - 1) // b * b


def _vsc_mesh():
    return plsc.VectorSubcoreMesh(core_axis_name="c", subcore_axis_name="s")


_SC_PARAMS = pltpu.CompilerParams(use_tc_tiling_on_sc=False)

_MASK_BITS = (
    "8a222eb193a459cdd7668e1a933c91e44ca8c361a99a316ed8f9c3e88cb12d8b"
    "5884d418566c9ac96c3f9aafa0fe2bb9431b6aebd58ff313fcde0029f1c7a40c"
    "cb52128792169864ad0b8cce369c436a0db1962ea572e625474ca1dc4bd605bb"
    "20285a1c84ddb8a209294a764110ba468e4a6d932361a9960db640071d18cd09"
    "83f755bc774658b85f7d41b0b618a5dac9b58898aed90ed419951b8a422ab5cc"
    "47f2a7042f49d543ed5b503f153ffd466f9e9c76367d6a2eb8ba5fb95f90fc4e"
    "46cabc443472c2f8644500ff2311148a27e39f974dba8d39f1287d2fff6c5464"
    "d9217b8360a827289199a56bc876727f1651bde6725a82dae8af4fc46681fd20"
)
_MASKF = np.unpackbits(
    np.frombuffer(bytes.fromhex(_MASK_BITS), dtype=np.uint8)
).reshape(_NG, 16).astype(np.float32)


def _sc_segment_sum(table, srcm, dstm, n_chunks_pad):
    nh, n, hd = table.shape
    rows_sub = _NODE_PAD // _NS
    trows = n // _NS
    assert n % _NS == 0
    col_split = nh == 2
    n_chunks_w = n_chunks_pad // (_NS if col_split else _NW)
    nbuf = 4
    iblk = 20
    assert n_chunks_w % iblk == 0 and iblk % nbuf == 0

    @functools.partial(
        pl.kernel,
        out_type=jax.ShapeDtypeStruct((_NC, _NODE_PAD, hd), jnp.float32),
        mesh=_vsc_mesh(),
        compiler_params=_SC_PARAMS,
        scratch_types=[
            pltpu.VMEM_SHARED((n, hd), jnp.float32),
            pltpu.VMEM_SHARED((_NODE_PAD, hd), jnp.float32),
            pltpu.VMEM((iblk, _CHUNK), jnp.int32),
            pltpu.VMEM((iblk, _CHUNK), jnp.int32),
        ]
        + [pltpu.VMEM((_CHUNK, hd), jnp.float32) for _ in range(nbuf)]
        + [pltpu.SemaphoreType.DMA for _ in range(nbuf)],
    )
    def seg_kernel(t_hbm, src_hbm, dst_hbm, out_hbm, table_sh, acc_sh,
                   srcs_v, dsts_v, *bufs_and_sems):
        rows = bufs_and_sems[:nbuf]
        sems = bufs_and_sems[nbuf:]
        c = lax.axis_index("c")
        s = lax.axis_index("s")
        base = s * rows_sub
        tidx = c if col_split else 0
        cbase = (s if col_split else c * _NS + s) * n_chunks_w

        pltpu.sync_copy(t_hbm.at[tidx, pl.ds(s * trows, trows)],
                        table_sh.at[pl.ds(s * trows, trows)])

        @pl.loop(0, _CHUNK)
        def _zrow(i):
            @pl.loop(0, hd // 16)
            def _zcol(j):
                rows[0][i, pl.ds(j * 16, 16)] = jnp.zeros((16,), jnp.float32)

        @pl.loop(0, rows_sub // _CHUNK)
        def _zacc(z):
            pltpu.sync_copy(rows[0],
                            acc_sh.at[pl.ds(base + z * _CHUNK, _CHUNK)])

        plsc.subcore_barrier()

        @pl.loop(0, n_chunks_w // iblk)
        def _blk(bi):
            pltpu.sync_copy(src_hbm.at[pl.ds(cbase + bi * iblk, iblk)],
                            srcs_v)
            pltpu.sync_copy(dst_hbm.at[pl.ds(cbase + bi * iblk, iblk)],
                            dsts_v)
            for b in range(nbuf):
                pltpu.async_copy(table_sh.at[srcs_v.at[b]], rows[b], sems[b])

            @pl.loop(0, iblk, step=nbuf)
            def _edges(j):
                for b in range(nbuf):
                    pltpu.make_async_copy(table_sh.at[srcs_v.at[b]],
                                          rows[b], sems[b]).wait()
                    pltpu.sync_copy(rows[b], acc_sh.at[dsts_v.at[j + b]],
                                    add=True)

                    @pl.when(j + b + nbuf < iblk)
                    def _refill():
                        pltpu.async_copy(
                            table_sh.at[srcs_v.at[j + b + nbuf]],
                            rows[b], sems[b])

        plsc.subcore_barrier()
        pltpu.sync_copy(acc_sh.at[pl.ds(base, rows_sub)],
                        out_hbm.at[c, pl.ds(base, rows_sub)])

    return seg_kernel(table, srcm, dstm)


def _sc_degree(dstm, n_chunks_w):
    @functools.partial(
        pl.kernel,
        out_type=jax.ShapeDtypeStruct((_NC, _NODE_PAD, 16), jnp.float32),
        mesh=_vsc_mesh(),
        compiler_params=_SC_PARAMS,
        scratch_types=[
            pltpu.VMEM_SHARED((_NODE_PAD, 16), jnp.float32),
            pltpu.VMEM((n_chunks_w, _CHUNK), jnp.int32),
            pltpu.VMEM((_CHUNK, 16), jnp.float32),
            pltpu.VMEM((_CHUNK, 16), jnp.float32),
        ],
    )
    def deg_kernel(dst_hbm, out_hbm, acc_sh, dsts_v, ones_v, zero_v):
        c = lax.axis_index("c")
        s = lax.axis_index("s")
        wid = c * _NS + s
        rows_sub = _NODE_PAD // _NS

        @pl.loop(0, _CHUNK)
        def _fill(i):
            ones_v[i, pl.ds(0, 16)] = jnp.ones((16,), jnp.float32)
            zero_v[i, pl.ds(0, 16)] = jnp.zeros((16,), jnp.float32)

        base = s * rows_sub

        @pl.loop(0, rows_sub // _CHUNK)
        def _zacc(k):
            pltpu.sync_copy(zero_v, acc_sh.at[pl.ds(base + k * _CHUNK, _CHUNK)])

        cbase = wid * n_chunks_w
        pltpu.sync_copy(dst_hbm.at[pl.ds(cbase, n_chunks_w)], dsts_v)
        plsc.subcore_barrier()

        @pl.loop(0, n_chunks_w)
        def _edges(j):
            pltpu.sync_copy(ones_v, acc_sh.at[dsts_v.at[j]], add=True)

        plsc.subcore_barrier()
        pltpu.sync_copy(acc_sh.at[pl.ds(base, rows_sub)],
                        out_hbm.at[c, pl.ds(base, rows_sub)])

    return deg_kernel(dstm)


def _tc_matmul(x, w):
    m, k = x.shape
    n = w.shape[1]
    bm = 1000

    def mm_kernel(x_ref, w_ref, o_ref):
        o_ref[...] = jnp.dot(x_ref[...], w_ref[...],
                             preferred_element_type=jnp.float32, precision=_HIGH)

    return pl.pallas_call(
        mm_kernel,
        grid=(m // bm,),
        in_specs=[pl.BlockSpec((bm, k), lambda i: (i, 0)),
                  pl.BlockSpec((k, n), lambda i: (0, 0))],
        out_specs=pl.BlockSpec((bm, n), lambda i: (i, 0)),
        out_shape=jax.ShapeDtypeStruct((m, n), jnp.float32),
    )(x, w)


def _tc_layer1(acc, deg, r, wl, b, wnext):
    m, dh = r.shape
    hd = acc.shape[2]
    dn = wnext.shape[1]
    bm = 1000

    def k(acc_ref, deg_ref, r_ref, wl_ref, b_ref, wn_ref, h_ref, p_ref):
        a = jnp.concatenate([acc_ref[0], acc_ref[1]], axis=1)
        dg = deg_ref[0][:, 0:1] + deg_ref[1][:, 0:1]
        mean = a / jnp.maximum(dg, 1.0)
        h = jnp.maximum(
            jnp.dot(mean, wl_ref[...], preferred_element_type=jnp.float32,
                    precision=_HIGH) + r_ref[...] + b_ref[...], 0.0)
        h_ref[...] = h
        p = jnp.dot(h, wn_ref[...], preferred_element_type=jnp.float32,
                    precision=_HIGH)
        p_ref[0] = p[:, :dn // 2]
        p_ref[1] = p[:, dn // 2:]

    return pl.pallas_call(
        k,
        grid=(m // bm,),
        in_specs=[pl.BlockSpec((_NC, bm, hd), lambda i: (0, i, 0)),
                  pl.BlockSpec((_NC, bm, 16), lambda i: (0, i, 0)),
                  pl.BlockSpec((bm, dh), lambda i: (i, 0)),
                  pl.BlockSpec((2 * hd, dh), lambda i: (0, 0)),
                  pl.BlockSpec((1, dh), lambda i: (0, 0)),
                  pl.BlockSpec((dh, dn), lambda i: (0, 0))],
        out_specs=[pl.BlockSpec((bm, dh), lambda i: (i, 0)),
                   pl.BlockSpec((2, bm, dn // 2), lambda i: (0, i, 0))],
        out_shape=[jax.ShapeDtypeStruct((m, dh), jnp.float32),
                   jax.ShapeDtypeStruct((2, m, dn // 2), jnp.float32)],
    )(acc, deg, r, wl, b, wnext)


def _tc_layer_pre(acc, deg, r, b, wnext):
    m, dh = r.shape
    hd = acc.shape[2]
    dn = wnext.shape[1]
    bm = 1000

    def k(acc_ref, deg_ref, r_ref, b_ref, wn_ref, h_ref, p_ref):
        a = jnp.concatenate([acc_ref[0], acc_ref[1]], axis=1)
        dg = deg_ref[0][:, 0:1] + deg_ref[1][:, 0:1]
        h = jnp.maximum(a / jnp.maximum(dg, 1.0) + r_ref[...] + b_ref[...], 0.0)
        h_ref[...] = h
        p_ref[...] = jnp.dot(h, wn_ref[...], preferred_element_type=jnp.float32,
                             precision=_HIGH)

    return pl.pallas_call(
        k,
        grid=(m // bm,),
        in_specs=[pl.BlockSpec((_NC, bm, hd), lambda i: (0, i, 0)),
                  pl.BlockSpec((_NC, bm, 16), lambda i: (0, i, 0)),
                  pl.BlockSpec((bm, dh), lambda i: (i, 0)),
                  pl.BlockSpec((1, dh), lambda i: (0, 0)),
                  pl.BlockSpec((dh, dn), lambda i: (0, 0))],
        out_specs=[pl.BlockSpec((bm, dh), lambda i: (i, 0)),
                   pl.BlockSpec((bm, dn), lambda i: (i, 0))],
        out_shape=[jax.ShapeDtypeStruct((m, dh), jnp.float32),
                   jax.ShapeDtypeStruct((m, dn), jnp.float32)],
    )(acc, deg, r, b, wnext)


def _tc_final(acc, deg, r, b, batch2d, maskf, wc, bc):
    m, dh = r.shape

    def k(acc_ref, deg_ref, r_ref, b_ref, bt_ref, mk_ref, wc_ref, bc_ref,
          out_ref, hd_ref):
        a = acc_ref[0] + acc_ref[1]
        dg = deg_ref[0][:, 0:1] + deg_ref[1][:, 0:1]
        h3 = a / jnp.maximum(dg, 1.0) + r_ref[...] + b_ref[...]
        oh = (lax.broadcasted_iota(jnp.int32, (_NG, m), 0)
              == bt_ref[...]).astype(jnp.float32)
        h3aug = jnp.concatenate([h3, jnp.ones((m, dh), jnp.float32)], axis=1)
        pooled = jnp.dot(oh, h3aug, preferred_element_type=jnp.float32,
                         precision=_HIGH)
        counts = pooled[:, dh:dh + 1]
        hp = pooled[:, :dh] / jnp.maximum(counts, 1.0)
        hd = hp * mk_ref[...] * 2.0
        hd_ref[...] = hd
        out_ref[...] = jnp.dot(hd, wc_ref[...], preferred_element_type=jnp.float32,
                               precision=_HIGH) + bc_ref[...]

    return pl.pallas_call(
        k,
        out_shape=[jax.ShapeDtypeStruct((_NG, 1), jnp.float32),
                   jax.ShapeDtypeStruct((_NG, dh), jnp.float32)],
    )(acc, deg, r, b, batch2d, maskf, wc, bc)


def kernel(x, edge_index, batch, Wl1, Wr1, b1, Wl2, Wr2, b2, Wl3, Wr3, b3, Wc, bc):
    n = x.shape[0]
    e = edge_index.shape[1]
    n_chunks_pad = _round_up(_round_up(e, _CHUNK) // _CHUNK, _NW * 4)
    n_chunks_w = n_chunks_pad // _NW
    pad = n_chunks_pad * _CHUNK - e
    pad_src = np.zeros((pad,), np.int32)
    pad_dst = n + np.arange(pad, dtype=np.int32) % (_NODE_PAD - n)
    src = jnp.concatenate(
        [edge_index[0].astype(jnp.int32), pad_src]).reshape(n_chunks_pad, _CHUNK)
    dst = jnp.concatenate(
        [edge_index[1].astype(jnp.int32), pad_dst]).reshape(n_chunks_pad, _CHUNK)

    deg2 = _sc_degree(dst, n_chunks_w)
    r1 = _tc_matmul(x, Wr1)
    x_st = jnp.stack([x[:, :64], x[:, 64:]])
    acc1 = _sc_segment_sum(x_st, src, dst, n_chunks_pad)
    h1, p2 = _tc_layer1(acc1, deg2, r1, Wl1, b1.reshape(1, -1), Wl2)
    r2 = _tc_matmul(h1, Wr2)
    acc2 = _sc_segment_sum(p2, src, dst, n_chunks_pad)
    h2, p3 = _tc_layer_pre(acc2, deg2, r2, b2.reshape(1, -1), Wl3)
    r3 = _tc_matmul(h2, Wr3)
    acc3 = _sc_segment_sum(p3[None], src, dst, n_chunks_pad)

    out2, hd = _tc_final(acc3[:, :n], deg2[:, :n], r3, b3.reshape(1, -1),
                         batch.astype(jnp.int32).reshape(1, -1),
                         jnp.asarray(_MASKF), Wc, bc.reshape(1, 1))
    return (out2.reshape(-1), hd)


# speedup vs baseline: 1.1125x; 1.0607x over previous
import functools

import jax
import jax.numpy as jnp
import numpy as np
from jax import lax
from jax.experimental import pallas as pl
from jax.experimental.pallas import tpu as pltpu
from jax.experimental.pallas import tpu_sc as plsc

_NC = 2
_NS = 16
_NW = _NC * _NS
_CHUNK = 128
_NODE_PAD = 10240
_NG = 128
_HIGH = lax.Precision.HIGHEST


def _round_up(a, b):
    return (a + b - 1) // b * b


def _vsc_mesh():
    return plsc.VectorSubcoreMesh(core_axis_name="c", subcore_axis_name="s")


_SC_PARAMS = pltpu.CompilerParams(use_tc_tiling_on_sc=False)

_MASK_BITS = (
    "8a222eb193a459cdd7668e1a933c91e44ca8c361a99a316ed8f9c3e88cb12d8b"
    "5884d418566c9ac96c3f9aafa0fe2bb9431b6aebd58ff313fcde0029f1c7a40c"
    "cb52128792169864ad0b8cce369c436a0db1962ea572e625474ca1dc4bd605bb"
    "20285a1c84ddb8a209294a764110ba468e4a6d932361a9960db640071d18cd09"
    "83f755bc774658b85f7d41b0b618a5dac9b58898aed90ed419951b8a422ab5cc"
    "47f2a7042f49d543ed5b503f153ffd466f9e9c76367d6a2eb8ba5fb95f90fc4e"
    "46cabc443472c2f8644500ff2311148a27e39f974dba8d39f1287d2fff6c5464"
    "d9217b8360a827289199a56bc876727f1651bde6725a82dae8af4fc46681fd20"
)
_MASKF = np.unpackbits(
    np.frombuffer(bytes.fromhex(_MASK_BITS), dtype=np.uint8)
).reshape(_NG, 16).astype(np.float32)


def _sc_segment_sum(table, srcm, dstm, n_chunks_pad):
    nh, n, hd = table.shape
    rows_sub = _NODE_PAD // _NS
    trows = n // _NS
    assert n % _NS == 0
    col_split = nh == 2
    n_chunks_w = n_chunks_pad // (_NS if col_split else _NW)
    nbuf = 4
    iblk = 40
    assert n_chunks_w % iblk == 0 and iblk % nbuf == 0

    @functools.partial(
        pl.kernel,
        out_type=jax.ShapeDtypeStruct((_NC, _NODE_PAD, hd), jnp.float32),
        mesh=_vsc_mesh(),
        compiler_params=_SC_PARAMS,
        scratch_types=[
            pltpu.VMEM_SHARED((n, hd), jnp.float32),
            pltpu.VMEM_SHARED((_NODE_PAD, hd), jnp.float32),
            pltpu.VMEM((iblk, _CHUNK), jnp.int32),
            pltpu.VMEM((iblk, _CHUNK), jnp.int32),
        ]
        + [pltpu.VMEM((_CHUNK, hd), jnp.float32) for _ in range(nbuf)]
        + [pltpu.SemaphoreType.DMA for _ in range(nbuf)],
    )
    def seg_kernel(t_hbm, src_hbm, dst_hbm, out_hbm, table_sh, acc_sh,
                   srcs_v, dsts_v, *bufs_and_sems):
        rows = bufs_and_sems[:nbuf]
        sems = bufs_and_sems[nbuf:]
        c = lax.axis_index("c")
        s = lax.axis_index("s")
        base = s * rows_sub
        tidx = c if col_split else 0
        cbase = (s if col_split else c * _NS + s) * n_chunks_w

        pltpu.sync_copy(t_hbm.at[tidx, pl.ds(s * trows, trows)],
                        table_sh.at[pl.ds(s * trows, trows)])

        @pl.loop(0, _CHUNK)
        def _zrow(i):
            @pl.loop(0, hd // 16)
            def _zcol(j):
                rows[0][i, pl.ds(j * 16, 16)] = jnp.zeros((16,), jnp.float32)

        @pl.loop(0, rows_sub // _CHUNK)
        def _zacc(z):
            pltpu.sync_copy(rows[0],
                            acc_sh.at[pl.ds(base + z * _CHUNK, _CHUNK)])

        plsc.subcore_barrier()

        @pl.loop(0, n_chunks_w // iblk)
        def _blk(bi):
            pltpu.sync_copy(src_hbm.at[pl.ds(cbase + bi * iblk, iblk)],
                            srcs_v)
            pltpu.sync_copy(dst_hbm.at[pl.ds(cbase + bi * iblk, iblk)],
                            dsts_v)
            for b in range(nbuf):
                pltpu.async_copy(table_sh.at[srcs_v.at[b]], rows[b], sems[b])

            @pl.loop(0, iblk, step=nbuf)
            def _edges(j):
                for b in range(nbuf):
                    pltpu.make_async_copy(table_sh.at[srcs_v.at[b]],
                                          rows[b], sems[b]).wait()
                    pltpu.sync_copy(rows[b], acc_sh.at[dsts_v.at[j + b]],
                                    add=True)

                    @pl.when(j + b + nbuf < iblk)
                    def _refill():
                        pltpu.async_copy(
                            table_sh.at[srcs_v.at[j + b + nbuf]],
                            rows[b], sems[b])

        plsc.subcore_barrier()
        pltpu.sync_copy(acc_sh.at[pl.ds(base, rows_sub)],
                        out_hbm.at[c, pl.ds(base, rows_sub)])

    return seg_kernel(table, srcm, dstm)


def _sc_degree(dstm, n_chunks_w):
    @functools.partial(
        pl.kernel,
        out_type=jax.ShapeDtypeStruct((_NC, _NODE_PAD, 16), jnp.float32),
        mesh=_vsc_mesh(),
        compiler_params=_SC_PARAMS,
        scratch_types=[
            pltpu.VMEM_SHARED((_NODE_PAD, 16), jnp.float32),
            pltpu.VMEM((n_chunks_w, _CHUNK), jnp.int32),
            pltpu.VMEM((_CHUNK, 16), jnp.float32),
            pltpu.VMEM((_CHUNK, 16), jnp.float32),
        ],
    )
    def deg_kernel(dst_hbm, out_hbm, acc_sh, dsts_v, ones_v, zero_v):
        c = lax.axis_index("c")
        s = lax.axis_index("s")
        wid = c * _NS + s
        rows_sub = _NODE_PAD // _NS

        @pl.loop(0, _CHUNK)
        def _fill(i):
            ones_v[i, pl.ds(0, 16)] = jnp.ones((16,), jnp.float32)
            zero_v[i, pl.ds(0, 16)] = jnp.zeros((16,), jnp.float32)

        base = s * rows_sub

        @pl.loop(0, rows_sub // _CHUNK)
        def _zacc(k):
            pltpu.sync_copy(zero_v, acc_sh.at[pl.ds(base + k * _CHUNK, _CHUNK)])

        cbase = wid * n_chunks_w
        pltpu.sync_copy(dst_hbm.at[pl.ds(cbase, n_chunks_w)], dsts_v)
        plsc.subcore_barrier()

        @pl.loop(0, n_chunks_w)
        def _edges(j):
            pltpu.sync_copy(ones_v, acc_sh.at[dsts_v.at[j]], add=True)

        plsc.subcore_barrier()
        pltpu.sync_copy(acc_sh.at[pl.ds(base, rows_sub)],
                        out_hbm.at[c, pl.ds(base, rows_sub)])

    return deg_kernel(dstm)


def _tc_matmul(x, w):
    m, k = x.shape
    n = w.shape[1]
    bm = 1000

    def mm_kernel(x_ref, w_ref, o_ref):
        o_ref[...] = jnp.dot(x_ref[...], w_ref[...],
                             preferred_element_type=jnp.float32, precision=_HIGH)

    return pl.pallas_call(
        mm_kernel,
        grid=(m // bm,),
        in_specs=[pl.BlockSpec((bm, k), lambda i: (i, 0)),
                  pl.BlockSpec((k, n), lambda i: (0, 0))],
        out_specs=pl.BlockSpec((bm, n), lambda i: (i, 0)),
        out_shape=jax.ShapeDtypeStruct((m, n), jnp.float32),
    )(x, w)


def _tc_layer1(acc, deg, r, wl, b, wnext):
    m, dh = r.shape
    hd = acc.shape[2]
    dn = wnext.shape[1]
    bm = 1000

    def k(acc_ref, deg_ref, r_ref, wl_ref, b_ref, wn_ref, h_ref, p_ref):
        a = jnp.concatenate([acc_ref[0], acc_ref[1]], axis=1)
        dg = deg_ref[0][:, 0:1] + deg_ref[1][:, 0:1]
        mean = a / jnp.maximum(dg, 1.0)
        h = jnp.maximum(
            jnp.dot(mean, wl_ref[...], preferred_element_type=jnp.float32,
                    precision=_HIGH) + r_ref[...] + b_ref[...], 0.0)
        h_ref[...] = h
        p = jnp.dot(h, wn_ref[...], preferred_element_type=jnp.float32,
                    precision=_HIGH)
        p_ref[0] = p[:, :dn // 2]
        p_ref[1] = p[:, dn // 2:]

    return pl.pallas_call(
        k,
        grid=(m // bm,),
        in_specs=[pl.BlockSpec((_NC, bm, hd), lambda i: (0, i, 0)),
                  pl.BlockSpec((_NC, bm, 16), lambda i: (0, i, 0)),
                  pl.BlockSpec((bm, dh), lambda i: (i, 0)),
                  pl.BlockSpec((2 * hd, dh), lambda i: (0, 0)),
                  pl.BlockSpec((1, dh), lambda i: (0, 0)),
                  pl.BlockSpec((dh, dn), lambda i: (0, 0))],
        out_specs=[pl.BlockSpec((bm, dh), lambda i: (i, 0)),
                   pl.BlockSpec((2, bm, dn // 2), lambda i: (0, i, 0))],
        out_shape=[jax.ShapeDtypeStruct((m, dh), jnp.float32),
                   jax.ShapeDtypeStruct((2, m, dn // 2), jnp.float32)],
    )(acc, deg, r, wl, b, wnext)


def _tc_layer_pre(acc, deg, r, b, wnext):
    m, dh = r.shape
    hd = acc.shape[2]
    dn = wnext.shape[1]
    bm = 1000

    def k(acc_ref, deg_ref, r_ref, b_ref, wn_ref, h_ref, p_ref):
        a = jnp.concatenate([acc_ref[0], acc_ref[1]], axis=1)
        dg = deg_ref[0][:, 0:1] + deg_ref[1][:, 0:1]
        h = jnp.maximum(a / jnp.maximum(dg, 1.0) + r_ref[...] + b_ref[...], 0.0)
        h_ref[...] = h
        p_ref[...] = jnp.dot(h, wn_ref[...], preferred_element_type=jnp.float32,
                             precision=_HIGH)

    return pl.pallas_call(
        k,
        grid=(m // bm,),
        in_specs=[pl.BlockSpec((_NC, bm, hd), lambda i: (0, i, 0)),
                  pl.BlockSpec((_NC, bm, 16), lambda i: (0, i, 0)),
                  pl.BlockSpec((bm, dh), lambda i: (i, 0)),
                  pl.BlockSpec((1, dh), lambda i: (0, 0)),
                  pl.BlockSpec((dh, dn), lambda i: (0, 0))],
        out_specs=[pl.BlockSpec((bm, dh), lambda i: (i, 0)),
                   pl.BlockSpec((bm, dn), lambda i: (i, 0))],
        out_shape=[jax.ShapeDtypeStruct((m, dh), jnp.float32),
                   jax.ShapeDtypeStruct((m, dn), jnp.float32)],
    )(acc, deg, r, b, wnext)


def _tc_final(acc, deg, r, b, batch2d, maskf, wc, bc):
    m, dh = r.shape

    def k(acc_ref, deg_ref, r_ref, b_ref, bt_ref, mk_ref, wc_ref, bc_ref,
          out_ref, hd_ref):
        a = acc_ref[0] + acc_ref[1]
        dg = deg_ref[0][:, 0:1] + deg_ref[1][:, 0:1]
        h3 = a / jnp.maximum(dg, 1.0) + r_ref[...] + b_ref[...]
        oh = (lax.broadcasted_iota(jnp.int32, (_NG, m), 0)
              == bt_ref[...]).astype(jnp.float32)
        h3aug = jnp.concatenate([h3, jnp.ones((m, dh), jnp.float32)], axis=1)
        pooled = jnp.dot(oh, h3aug, preferred_element_type=jnp.float32,
                         precision=_HIGH)
        counts = pooled[:, dh:dh + 1]
        hp = pooled[:, :dh] / jnp.maximum(counts, 1.0)
        hd = hp * mk_ref[...] * 2.0
        hd_ref[...] = hd
        out_ref[...] = jnp.dot(hd, wc_ref[...], preferred_element_type=jnp.float32,
                               precision=_HIGH) + bc_ref[...]

    return pl.pallas_call(
        k,
        out_shape=[jax.ShapeDtypeStruct((_NG, 1), jnp.float32),
                   jax.ShapeDtypeStruct((_NG, dh), jnp.float32)],
    )(acc, deg, r, b, batch2d, maskf, wc, bc)


def kernel(x, edge_index, batch, Wl1, Wr1, b1, Wl2, Wr2, b2, Wl3, Wr3, b3, Wc, bc):
    n = x.shape[0]
    e = edge_index.shape[1]
    n_chunks_pad = _round_up(_round_up(e, _CHUNK) // _CHUNK, _NW * 4)
    n_chunks_w = n_chunks_pad // _NW
    pad = n_chunks_pad * _CHUNK - e
    pad_src = np.zeros((pad,), np.int32)
    pad_dst = n + np.arange(pad, dtype=np.int32) % (_NODE_PAD - n)
    src = jnp.concatenate(
        [edge_index[0].astype(jnp.int32), pad_src]).reshape(n_chunks_pad, _CHUNK)
    dst = jnp.concatenate(
        [edge_index[1].astype(jnp.int32), pad_dst]).reshape(n_chunks_pad, _CHUNK)

    deg2 = _sc_degree(dst, n_chunks_w)
    r1 = _tc_matmul(x, Wr1)
    x_st = jnp.stack([x[:, :64], x[:, 64:]])
    acc1 = _sc_segment_sum(x_st, src, dst, n_chunks_pad)
    h1, p2 = _tc_layer1(acc1, deg2, r1, Wl1, b1.reshape(1, -1), Wl2)
    r2 = _tc_matmul(h1, Wr2)
    acc2 = _sc_segment_sum(p2, src, dst, n_chunks_pad)
    h2, p3 = _tc_layer_pre(acc2, deg2, r2, b2.reshape(1, -1), Wl3)
    r3 = _tc_matmul(h2, Wr3)
    acc3 = _sc_segment_sum(p3[None], src, dst, n_chunks_pad)

    out2, hd = _tc_final(acc3[:, :n], deg2[:, :n], r3, b3.reshape(1, -1),
                         batch.astype(jnp.int32).reshape(1, -1),
                         jnp.asarray(_MASKF), Wc, bc.reshape(1, 1))
    return (out2.reshape(-1), hd)
